# baseline (device time: 45962 ns/iter reference)
import jax
import jax.numpy as jnp
from jax import lax
from jax.experimental import pallas as pl
from jax.experimental.pallas import tpu as pltpu

N_DEV = 8
BR = 64
D = 512
H = 1024


def kernel(x, Win0, Wout0, Win1, Wout1, Win2, Wout2):
    def body(x_ref, win0_ref, wout0_ref, win1_ref, wout1_ref, win2_ref,
             wout2_ref, out_ref, xg_ref, part_ref, acc_ref, xcur_ref,
             xb_ref, pb_ref, winb_ref, woutb_ref, b_send, b_recv,
             r_send, r_recv):
        my = lax.axis_index("i")

        def peer(o):
            return lax.rem(my + o, N_DEV)

        barrier_sem = pltpu.get_barrier_semaphore()
        for o in range(1, N_DEV):
            pl.semaphore_signal(
                barrier_sem, inc=1,
                device_id=(peer(o),), device_id_type=pl.DeviceIdType.MESH,
            )
        pl.semaphore_wait(barrier_sem, N_DEV - 1)

        xcur_ref[...] = x_ref[...]
        acc_ref[my] = jnp.zeros((BR, D), jnp.bfloat16)

        weights = (
            (win0_ref, wout0_ref),
            (win1_ref, wout1_ref),
            (win2_ref, wout2_ref),
        )
        for win_ref, wout_ref in weights:
            xb_ref[...] = xcur_ref[...].astype(jnp.bfloat16)
            winb_ref[...] = win_ref[...].astype(jnp.bfloat16)
            woutb_ref[...] = wout_ref[...].astype(jnp.bfloat16)
            bcasts = []
            for o in range(1, N_DEV):
                rdma = pltpu.make_async_remote_copy(
                    src_ref=xb_ref,
                    dst_ref=xg_ref.at[pl.ds(my * BR, BR), :],
                    send_sem=b_send.at[o],
                    recv_sem=b_recv.at[my],
                    device_id=(peer(o),),
                    device_id_type=pl.DeviceIdType.MESH,
                )
                rdma.start()
                bcasts.append(rdma)

            def wait_x_from(s):
                recv = pltpu.make_async_remote_copy(
                    src_ref=xb_ref,
                    dst_ref=xg_ref.at[pl.ds(s * BR, BR), :],
                    send_sem=b_send.at[0],
                    recv_sem=b_recv.at[s],
                    device_id=(my,),
                    device_id_type=pl.DeviceIdType.MESH,
                )
                recv.wait_recv()

            reds = []
            for oa, ob in ((0, 7), (6, 5), (4, 3), (2, 1)):
                rows = []
                for o in (oa, ob):
                    if o == 0:
                        rows.append(xb_ref[...])
                    else:
                        s = peer(o)
                        wait_x_from(s)
                        rows.append(xg_ref[pl.ds(s * BR, BR), :])
                x2 = jnp.concatenate(rows, axis=0)
                h2 = jnp.maximum(
                    jnp.dot(x2, winb_ref[...],
                            preferred_element_type=jnp.float32),
                    0.0,
                ).astype(jnp.bfloat16)
                p2 = jnp.dot(h2, woutb_ref[...],
                             preferred_element_type=jnp.float32)
                for j, o in enumerate((oa, ob)):
                    pj = p2[j * BR:(j + 1) * BR, :]
                    if o == 0:
                        part_ref[...] = pj
                    else:
                        c = peer(o)
                        pb_ref[pl.ds(c * BR, BR), :] = pj.astype(jnp.bfloat16)
                        rdma = pltpu.make_async_remote_copy(
                            src_ref=pb_ref.at[pl.ds(c * BR, BR), :],
                            dst_ref=acc_ref.at[my],
                            send_sem=r_send.at[o],
                            recv_sem=r_recv.at[my],
                            device_id=(c,),
                            device_id_type=pl.DeviceIdType.MESH,
                        )
                        rdma.start()
                        reds.append(rdma)

            for o in range(1, N_DEV):
                s = peer(o)
                recv = pltpu.make_async_remote_copy(
                    src_ref=pb_ref.at[pl.ds(s * BR, BR), :],
                    dst_ref=acc_ref.at[s],
                    send_sem=r_send.at[0],
                    recv_sem=r_recv.at[s],
                    device_id=(my,),
                    device_id_type=pl.DeviceIdType.MESH,
                )
                recv.wait_recv()
            for rdma in bcasts:
                rdma.wait_send()
            for rdma in reds:
                rdma.wait_send()

            xcur_ref[...] = part_ref[...] + jnp.sum(
                acc_ref[...].astype(jnp.float32), axis=0
            )

        out_ref[...] = xcur_ref[...]

    return pl.pallas_call(
        body,
        out_shape=jax.ShapeDtypeStruct((BR, D), jnp.float32),
        in_specs=[pl.BlockSpec(memory_space=pltpu.VMEM)] * 7,
        out_specs=pl.BlockSpec(memory_space=pltpu.VMEM),
        scratch_shapes=[
            pltpu.VMEM((N_DEV * BR, D), jnp.bfloat16),
            pltpu.VMEM((BR, D), jnp.float32),
            pltpu.VMEM((N_DEV, BR, D), jnp.bfloat16),
            pltpu.VMEM((BR, D), jnp.float32),
            pltpu.VMEM((BR, D), jnp.bfloat16),
            pltpu.VMEM((N_DEV * BR, D), jnp.bfloat16),
            pltpu.VMEM((D, H), jnp.bfloat16),
            pltpu.VMEM((H, D), jnp.bfloat16),
            pltpu.SemaphoreType.DMA((N_DEV,)),
            pltpu.SemaphoreType.DMA((N_DEV,)),
            pltpu.SemaphoreType.DMA((N_DEV,)),
            pltpu.SemaphoreType.DMA((N_DEV,)),
        ],
        compiler_params=pltpu.CompilerParams(collective_id=0),
    )(x, Win0, Wout0, Win1, Wout1, Win2, Wout2)
